# baseline (speedup 1.0000x reference)
import jax
import jax.numpy as jnp
from jax.experimental import pallas as pl
from jax.experimental.pallas import tpu as pltpu

_NUM_CODE = 1024
_CODE_DIM = 64
_TB = 1024


def _vq_block(pi_ref, tg_ref, re_ref, ret_ref, out_ref, ren_ref, rent2_ref):
    eps = jnp.finfo(jnp.float32).eps

    @pl.when(jnp.logical_and(pl.program_id(0) == 0, pl.program_id(1) == 0))
    def _normalize_codebook():
        re = re_ref[...]
        n2 = jnp.sum(re * re, axis=1, keepdims=True) + eps
        ren = re / jnp.sqrt(n2)
        ci = jax.lax.broadcasted_iota(jnp.int32, (_NUM_CODE, _CODE_DIM), 1)
        ren_ref[...] = jnp.where(ci == 0, 0.0, ren)
        ret = ret_ref[...]
        n2t = jnp.sum(ret * ret, axis=0, keepdims=True) + eps
        rent = ret / jnp.sqrt(n2t)
        ri = jax.lax.broadcasted_iota(jnp.int32, (_CODE_DIM, _NUM_CODE), 0)
        rent2_ref[...] = jnp.where(ri == 0, 0.0, 2.0 * rent)

    pib = pi_ref[0]
    tgb = tg_ref[0]
    ren = ren_ref[...]

    def mm(a, b):
        return jax.lax.dot_general(
            a, b, (((1,), (0,)), ((), ())),
            preferred_element_type=jnp.float32,
            precision=jax.lax.Precision.DEFAULT)

    t1 = mm(ren, tgb)
    p = mm(ren, pib)
    ones = jnp.ones((1, _CODE_DIM), jnp.float32)
    rowdot = jax.lax.dot_general(
        ones, tgb * pib, (((1,), (0,)), ((), ())),
        preferred_element_type=jnp.float32,
        precision=jax.lax.Precision.HIGHEST)
    c = 2.0 - 2.0 * rowdot
    d = c + 4.0 * t1 * p

    dmin = jnp.min(d, axis=0, keepdims=True)
    iota = jax.lax.broadcasted_iota(jnp.int32, (_NUM_CODE, _TB), 0)
    idx = jnp.min(jnp.where(d == dmin, iota, _NUM_CODE),
                  axis=0, keepdims=True)
    s = jnp.where(iota == idx, p, 0.0)

    out_ref[0] = pib - mm(rent2_ref[...], s)


def kernel(prev_input, target, rot_emb):
    B, N, T = prev_input.shape
    grid = (B, T // _TB)
    return pl.pallas_call(
        _vq_block,
        grid=grid,
        in_specs=[
            pl.BlockSpec((1, N, _TB), lambda b, t: (b, 0, t)),
            pl.BlockSpec((1, N, _TB), lambda b, t: (b, 0, t)),
            pl.BlockSpec((_NUM_CODE, _CODE_DIM), lambda b, t: (0, 0)),
            pl.BlockSpec((_CODE_DIM, _NUM_CODE), lambda b, t: (0, 0)),
        ],
        out_specs=pl.BlockSpec((1, N, _TB), lambda b, t: (b, 0, t)),
        out_shape=jax.ShapeDtypeStruct((B, N, T), jnp.float32),
        scratch_shapes=[
            pltpu.VMEM((_NUM_CODE, _CODE_DIM), jnp.float32),
            pltpu.VMEM((_CODE_DIM, _NUM_CODE), jnp.float32),
        ],
    )(prev_input, target, rot_emb, rot_emb.T)


# speedup vs baseline: 1.2842x; 1.2842x over previous
import jax
import jax.numpy as jnp
from jax.experimental import pallas as pl
from jax.experimental.pallas import tpu as pltpu

_NUM_CODE = 1024
_CODE_DIM = 64
_TB = 1024
_SPLIT = 4


def _vq_block(pi_ref, tg_ref, re_ref, ret_ref, out_ref, ren_ref, rent2_ref):
    eps = jnp.finfo(jnp.float32).eps

    @pl.when(jnp.logical_and(pl.program_id(0) == 0, pl.program_id(1) == 0))
    def _normalize_codebook():
        re = re_ref[...]
        n2 = jnp.sum(re * re, axis=1, keepdims=True) + eps
        ren = re / jnp.sqrt(n2)
        ci = jax.lax.broadcasted_iota(jnp.int32, (_NUM_CODE, _CODE_DIM), 1)
        ren_ref[...] = jnp.where(ci == 0, 0.0, ren)
        ret = ret_ref[...]
        n2t = jnp.sum(ret * ret, axis=0, keepdims=True) + eps
        rent = ret / jnp.sqrt(n2t)
        ri = jax.lax.broadcasted_iota(jnp.int32, (_CODE_DIM, _NUM_CODE), 0)
        rent2_ref[...] = jnp.where(ri == 0, 0.0, 2.0 * rent)

    ren = ren_ref[...]
    rent2 = rent2_ref[...]

    def mm(a, b):
        return jax.lax.dot_general(
            a, b, (((1,), (0,)), ((), ())),
            preferred_element_type=jnp.float32,
            precision=jax.lax.Precision.DEFAULT)

    cw = _TB // _SPLIT
    mms = []
    for h in range(_SPLIT):
        sl = slice(h * cw, (h + 1) * cw)
        pib = pi_ref[0, :, sl]
        tgb = tg_ref[0, :, sl]
        mms.append((sl, pib, tgb,
                    mm(ren, tgb),
                    mm(ren, pib)))
    for h in range(_SPLIT):
        sl, pib, tgb, t1, p = mms[h]
        tp = tgb * pib
        a0 = tp[0:8] + tp[8:16]
        a1 = tp[16:24] + tp[24:32]
        a2 = tp[32:40] + tp[40:48]
        a3 = tp[48:56] + tp[56:64]
        acc = (a0 + a1) + (a2 + a3)
        s4 = acc[0:4] + acc[4:8]
        s2 = s4[0:2] + s4[2:4]
        c = 2.0 - 2.0 * (s2[0:1] + s2[1:2])
        d = c + 4.0 * t1 * p

        dmin = jnp.min(d, axis=0, keepdims=True)
        iota = jax.lax.broadcasted_iota(jnp.int32, (_NUM_CODE, cw), 0)
        idx = jnp.min(jnp.where(d == dmin, iota, _NUM_CODE),
                      axis=0, keepdims=True)
        s = jnp.where(iota == idx, p, 0.0)

        out_ref[0, :, sl] = pib - mm(rent2, s)


def kernel(prev_input, target, rot_emb):
    B, N, T = prev_input.shape
    grid = (B, T // _TB)
    return pl.pallas_call(
        _vq_block,
        grid=grid,
        in_specs=[
            pl.BlockSpec((1, N, _TB), lambda b, t: (b, 0, t)),
            pl.BlockSpec((1, N, _TB), lambda b, t: (b, 0, t)),
            pl.BlockSpec((_NUM_CODE, _CODE_DIM), lambda b, t: (0, 0)),
            pl.BlockSpec((_CODE_DIM, _NUM_CODE), lambda b, t: (0, 0)),
        ],
        out_specs=pl.BlockSpec((1, N, _TB), lambda b, t: (b, 0, t)),
        out_shape=jax.ShapeDtypeStruct((B, N, T), jnp.float32),
        scratch_shapes=[
            pltpu.VMEM((_NUM_CODE, _CODE_DIM), jnp.float32),
            pltpu.VMEM((_CODE_DIM, _NUM_CODE), jnp.float32),
        ],
    )(prev_input, target, rot_emb, rot_emb.T)


# speedup vs baseline: 1.6532x; 1.2874x over previous
import jax
import jax.numpy as jnp
from jax.experimental import pallas as pl
from jax.experimental.pallas import tpu as pltpu

_NUM_CODE = 1024
_CODE_DIM = 64
_TB = 1024
_SPLIT = 4


def _vq_block(pi_ref, tg_ref, re_ref, ret_ref, out_ref, ren_ref, rent2_ref):
    eps = jnp.finfo(jnp.float32).eps

    @pl.when(jnp.logical_and(pl.program_id(0) == 0, pl.program_id(1) == 0))
    def _normalize_codebook():
        re = re_ref[...]
        n2 = jnp.sum(re * re, axis=1, keepdims=True) + eps
        ren = re / jnp.sqrt(n2)
        ci = jax.lax.broadcasted_iota(jnp.int32, (_NUM_CODE, _CODE_DIM), 1)
        ren_ref[...] = jnp.where(ci == 0, 0.0, ren)
        ret = ret_ref[...]
        n2t = jnp.sum(ret * ret, axis=0, keepdims=True) + eps
        rent = ret / jnp.sqrt(n2t)
        ri = jax.lax.broadcasted_iota(jnp.int32, (_CODE_DIM, _NUM_CODE), 0)
        rent2_ref[...] = jnp.where(ri == 0, 0.0, 2.0 * rent)

    ren = ren_ref[...]
    rent2 = rent2_ref[...]

    def mm(a, b):
        return jax.lax.dot_general(
            a, b, (((1,), (0,)), ((), ())),
            preferred_element_type=jnp.float32,
            precision=jax.lax.Precision.DEFAULT)

    cw = _TB // _SPLIT
    mms = []
    for h in range(_SPLIT):
        sl = slice(h * cw, (h + 1) * cw)
        pib = pi_ref[0, :, sl]
        tgb = tg_ref[0, :, sl]
        mms.append((sl, pib, tgb,
                    mm(ren, tgb),
                    mm(ren, pib)))
    for h in range(_SPLIT):
        sl, pib, tgb, t1, p = mms[h]
        tp = tgb * pib
        a0 = tp[0:8] + tp[8:16]
        a1 = tp[16:24] + tp[24:32]
        a2 = tp[32:40] + tp[40:48]
        a3 = tp[48:56] + tp[56:64]
        acc = (a0 + a1) + (a2 + a3)
        s4 = acc[0:4] + acc[4:8]
        s2 = s4[0:2] + s4[2:4]
        c = 2.0 - 2.0 * (s2[0:1] + s2[1:2])

        m = jnp.full((8, cw), jnp.inf, jnp.float32)
        mi = jnp.zeros((8, cw), jnp.int32)
        for k in range(_NUM_CODE // 8):
            rs = slice(8 * k, 8 * k + 8)
            dk = c + 4.0 * t1[rs] * p[rs]
            upd = dk < m
            m = jnp.where(upd, dk, m)
            mi = jnp.where(upd, k, mi)
        jj = jax.lax.broadcasted_iota(jnp.int32, (8, cw), 0)
        cc = mi * 8 + jj
        for w in (4, 2, 1):
            v1, v2 = m[:w], m[w:2 * w]
            c1, c2 = cc[:w], cc[w:2 * w]
            take2 = (v2 < v1) | ((v2 == v1) & (c2 < c1))
            m = jnp.where(take2, v2, v1)
            cc = jnp.where(take2, c2, c1)
        idx = cc
        iota = jax.lax.broadcasted_iota(jnp.int32, (_NUM_CODE, cw), 0)
        s = jnp.where(iota == idx, p, 0.0)

        out_ref[0, :, sl] = pib - mm(rent2, s)


def kernel(prev_input, target, rot_emb):
    B, N, T = prev_input.shape
    grid = (B, T // _TB)
    return pl.pallas_call(
        _vq_block,
        grid=grid,
        in_specs=[
            pl.BlockSpec((1, N, _TB), lambda b, t: (b, 0, t)),
            pl.BlockSpec((1, N, _TB), lambda b, t: (b, 0, t)),
            pl.BlockSpec((_NUM_CODE, _CODE_DIM), lambda b, t: (0, 0)),
            pl.BlockSpec((_CODE_DIM, _NUM_CODE), lambda b, t: (0, 0)),
        ],
        out_specs=pl.BlockSpec((1, N, _TB), lambda b, t: (b, 0, t)),
        out_shape=jax.ShapeDtypeStruct((B, N, T), jnp.float32),
        scratch_shapes=[
            pltpu.VMEM((_NUM_CODE, _CODE_DIM), jnp.float32),
            pltpu.VMEM((_CODE_DIM, _NUM_CODE), jnp.float32),
        ],
    )(prev_input, target, rot_emb, rot_emb.T)


# speedup vs baseline: 1.7654x; 1.0678x over previous
import jax
import jax.numpy as jnp
from jax.experimental import pallas as pl
from jax.experimental.pallas import tpu as pltpu

_NUM_CODE = 1024
_CODE_DIM = 64
_TB = 1024
_SPLIT = 4
_BB = 2


def _vq_block(pi_ref, tg_ref, re_ref, ret_ref, out_ref, ren2_ref, rent_ref):
    eps = jnp.finfo(jnp.float32).eps

    @pl.when(jnp.logical_and(pl.program_id(0) == 0, pl.program_id(1) == 0))
    def _normalize_codebook():
        re = re_ref[...]
        n2 = jnp.sum(re * re, axis=1, keepdims=True) + eps
        ren = re / jnp.sqrt(n2)
        ci = jax.lax.broadcasted_iota(jnp.int32, (_NUM_CODE, _CODE_DIM), 1)
        ren2_ref[...] = jnp.where(ci == 0, 0.0, 2.0 * ren)
        ret = ret_ref[...]
        n2t = jnp.sum(ret * ret, axis=0, keepdims=True) + eps
        rent = ret / jnp.sqrt(n2t)
        ri = jax.lax.broadcasted_iota(jnp.int32, (_CODE_DIM, _NUM_CODE), 0)
        rent_ref[...] = jnp.where(ri == 0, 0.0, rent)

    ren2 = ren2_ref[...]
    rent = rent_ref[...]

    def mm(a, b):
        return jax.lax.dot_general(
            a, b, (((1,), (0,)), ((), ())),
            preferred_element_type=jnp.float32,
            precision=jax.lax.Precision.DEFAULT)

    cw = _TB // _SPLIT
    mms = []
    for bb in range(_BB):
        for h in range(_SPLIT):
            sl = slice(h * cw, (h + 1) * cw)
            pib = pi_ref[bb, :, sl]
            tgb = tg_ref[bb, :, sl]
            mms.append((bb, sl, pib, tgb,
                        mm(ren2, tgb),
                        mm(ren2, pib)))
    for bb, sl, pib, tgb, t1, p in mms:
        tp = tgb * pib
        a0 = tp[0:8] + tp[8:16]
        a1 = tp[16:24] + tp[24:32]
        a2 = tp[32:40] + tp[40:48]
        a3 = tp[48:56] + tp[56:64]
        acc = (a0 + a1) + (a2 + a3)
        s4 = acc[0:4] + acc[4:8]
        s2 = s4[0:2] + s4[2:4]
        c = 2.0 - 2.0 * (s2[0:1] + s2[1:2])

        m = jnp.full((8, cw), jnp.inf, jnp.float32)
        mi = jnp.zeros((8, cw), jnp.int32)
        for k in range(_NUM_CODE // 8):
            rs = slice(8 * k, 8 * k + 8)
            dk = c + t1[rs] * p[rs]
            upd = dk < m
            m = jnp.where(upd, dk, m)
            mi = jnp.where(upd, k, mi)
        jj = jax.lax.broadcasted_iota(jnp.int32, (8, cw), 0)
        cc = mi * 8 + jj
        for w in (4, 2, 1):
            v1, v2 = m[:w], m[w:2 * w]
            c1, c2 = cc[:w], cc[w:2 * w]
            take2 = (v2 < v1) | ((v2 == v1) & (c2 < c1))
            m = jnp.where(take2, v2, v1)
            cc = jnp.where(take2, c2, c1)
        idx = cc
        iota = jax.lax.broadcasted_iota(jnp.int32, (_NUM_CODE, cw), 0)
        s = jnp.where(iota == idx, p, 0.0)

        out_ref[bb, :, sl] = pib - mm(rent, s)


def kernel(prev_input, target, rot_emb):
    B, N, T = prev_input.shape
    grid = (B // _BB, T // _TB)
    return pl.pallas_call(
        _vq_block,
        grid=grid,
        in_specs=[
            pl.BlockSpec((_BB, N, _TB), lambda b, t: (b, 0, t)),
            pl.BlockSpec((_BB, N, _TB), lambda b, t: (b, 0, t)),
            pl.BlockSpec((_NUM_CODE, _CODE_DIM), lambda b, t: (0, 0)),
            pl.BlockSpec((_CODE_DIM, _NUM_CODE), lambda b, t: (0, 0)),
        ],
        out_specs=pl.BlockSpec((_BB, N, _TB), lambda b, t: (b, 0, t)),
        out_shape=jax.ShapeDtypeStruct((B, N, T), jnp.float32),
        scratch_shapes=[
            pltpu.VMEM((_NUM_CODE, _CODE_DIM), jnp.float32),
            pltpu.VMEM((_CODE_DIM, _NUM_CODE), jnp.float32),
        ],
    )(prev_input, target, rot_emb, rot_emb.T)


# speedup vs baseline: 1.8081x; 1.0242x over previous
import jax
import jax.numpy as jnp
from jax.experimental import pallas as pl
from jax.experimental.pallas import tpu as pltpu

_NUM_CODE = 1024
_CODE_DIM = 64
_TB = 1024
_SPLIT = 4
_BB = 4


def _vq_block(pi_ref, tg_ref, re_ref, ret_ref, out_ref, ren2_ref, rent_ref):
    eps = jnp.finfo(jnp.float32).eps

    @pl.when(jnp.logical_and(pl.program_id(0) == 0, pl.program_id(1) == 0))
    def _normalize_codebook():
        re = re_ref[...]
        n2 = jnp.sum(re * re, axis=1, keepdims=True) + eps
        ren = re / jnp.sqrt(n2)
        ci = jax.lax.broadcasted_iota(jnp.int32, (_NUM_CODE, _CODE_DIM), 1)
        ren2_ref[...] = jnp.where(ci == 0, 0.0, 2.0 * ren)
        ret = ret_ref[...]
        n2t = jnp.sum(ret * ret, axis=0, keepdims=True) + eps
        rent = ret / jnp.sqrt(n2t)
        ri = jax.lax.broadcasted_iota(jnp.int32, (_CODE_DIM, _NUM_CODE), 0)
        rent_ref[...] = jnp.where(ri == 0, 0.0, rent)

    ren2 = ren2_ref[...]
    rent = rent_ref[...]

    def mm(a, b):
        return jax.lax.dot_general(
            a, b, (((1,), (0,)), ((), ())),
            preferred_element_type=jnp.float32,
            precision=jax.lax.Precision.DEFAULT)

    cw = _TB // _SPLIT
    mms = []
    for bb in range(_BB):
        for h in range(_SPLIT):
            sl = slice(h * cw, (h + 1) * cw)
            pib = pi_ref[bb, :, sl]
            tgb = tg_ref[bb, :, sl]
            mms.append((bb, sl, pib, tgb,
                        mm(ren2, tgb),
                        mm(ren2, pib)))
    for bb, sl, pib, tgb, t1, p in mms:
        tp = tgb * pib
        a0 = tp[0:8] + tp[8:16]
        a1 = tp[16:24] + tp[24:32]
        a2 = tp[32:40] + tp[40:48]
        a3 = tp[48:56] + tp[56:64]
        acc = (a0 + a1) + (a2 + a3)
        s4 = acc[0:4] + acc[4:8]
        s2 = s4[0:2] + s4[2:4]
        c = 2.0 - 2.0 * (s2[0:1] + s2[1:2])

        m = jnp.full((8, cw), jnp.inf, jnp.float32)
        mi = jnp.zeros((8, cw), jnp.int32)
        for k in range(_NUM_CODE // 8):
            rs = slice(8 * k, 8 * k + 8)
            dk = c + t1[rs] * p[rs]
            upd = dk < m
            m = jnp.where(upd, dk, m)
            mi = jnp.where(upd, k, mi)
        jj = jax.lax.broadcasted_iota(jnp.int32, (8, cw), 0)
        cc = mi * 8 + jj
        for w in (4, 2, 1):
            v1, v2 = m[:w], m[w:2 * w]
            c1, c2 = cc[:w], cc[w:2 * w]
            take2 = (v2 < v1) | ((v2 == v1) & (c2 < c1))
            m = jnp.where(take2, v2, v1)
            cc = jnp.where(take2, c2, c1)
        idx = cc
        iota = jax.lax.broadcasted_iota(jnp.int32, (_NUM_CODE, cw), 0)
        s = jnp.where(iota == idx, p, 0.0)

        out_ref[bb, :, sl] = pib - mm(rent, s)


def kernel(prev_input, target, rot_emb):
    B, N, T = prev_input.shape
    grid = (B // _BB, T // _TB)
    return pl.pallas_call(
        _vq_block,
        grid=grid,
        in_specs=[
            pl.BlockSpec((_BB, N, _TB), lambda b, t: (b, 0, t)),
            pl.BlockSpec((_BB, N, _TB), lambda b, t: (b, 0, t)),
            pl.BlockSpec((_NUM_CODE, _CODE_DIM), lambda b, t: (0, 0)),
            pl.BlockSpec((_CODE_DIM, _NUM_CODE), lambda b, t: (0, 0)),
        ],
        out_specs=pl.BlockSpec((_BB, N, _TB), lambda b, t: (b, 0, t)),
        out_shape=jax.ShapeDtypeStruct((B, N, T), jnp.float32),
        scratch_shapes=[
            pltpu.VMEM((_NUM_CODE, _CODE_DIM), jnp.float32),
            pltpu.VMEM((_CODE_DIM, _NUM_CODE), jnp.float32),
        ],
    )(prev_input, target, rot_emb, rot_emb.T)
